# baseline (speedup 1.0000x reference)
import functools

import jax
import jax.numpy as jnp
from jax import lax
from jax.experimental import pallas as pl
from jax.experimental.pallas import tpu as pltpu
from jax.experimental.pallas import tpu_sc as plsc

N_NODES = 10000
D = 128
E = 320000
NW = 32
E_PER_W = E // NW
CHUNK = 100
N_CHUNKS = E_PER_W // CHUNK
N_STAGES = 2
SCHUNKS = N_CHUNKS // N_STAGES
N_PAD = 10240
STRIPE = N_PAD // 16
FCH = 80
FLUSH = STRIPE // FCH

_mesh = plsc.VectorSubcoreMesh(core_axis_name="c", subcore_axis_name="s")


@functools.partial(
    pl.kernel,
    out_type=jax.ShapeDtypeStruct((2 * N_PAD, D), jnp.float32),
    mesh=_mesh,
    scratch_types=[
        pltpu.VMEM((SCHUNKS, CHUNK), jnp.int32),
        pltpu.VMEM((SCHUNKS, CHUNK), jnp.int32),
        pltpu.VMEM((CHUNK, D), jnp.float32),
        pltpu.VMEM((CHUNK, D), jnp.float32),
        pltpu.VMEM_SHARED((N_PAD, D), jnp.float32),
        pltpu.SemaphoreType.DMA,
        pltpu.SemaphoreType.DMA,
        pltpu.SemaphoreType.DMA,
        pltpu.SemaphoreType.DMA,
    ],
)
def _gin_scatter(src_hbm, dst_hbm, feat_hbm, out_hbm,
                 sidx, didx, rows0, rows1, acc, sem0, sem1, sem2, sem3):
    cid = lax.axis_index("c")
    sid = lax.axis_index("s")
    wid = cid * 16 + sid

    zero = jnp.zeros((16,), jnp.float32)

    def zrow(r, _):
        def zcol(c, _):
            rows0[r, pl.ds(c * 16, 16)] = zero
            return ()
        lax.fori_loop(0, D // 16, zcol, ())
        return ()

    lax.fori_loop(0, FCH, zrow, ())

    def zflush(t, _):
        pltpu.sync_copy(rows0.at[pl.ds(0, FCH)],
                        acc.at[pl.ds(sid * STRIPE + t * FCH, FCH)])
        return ()

    lax.fori_loop(0, FLUSH, zflush, ())
    plsc.subcore_barrier()

    for s in range(N_STAGES):
        pltpu.sync_copy(src_hbm.at[wid, s], sidx)
        pltpu.sync_copy(dst_hbm.at[wid, s], didx)
        pltpu.async_copy(feat_hbm.at[sidx.at[0]], rows0, sem0)
        pltpu.async_copy(feat_hbm.at[sidx.at[1]], rows1, sem1)

        def body(j, _):
            c0 = 2 * j
            c1 = c0 + 1
            pltpu.make_async_copy(feat_hbm.at[sidx.at[c0]], rows0, sem0).wait()
            pltpu.async_copy(rows0, acc.at[didx.at[c0]], sem2, add=True)
            pltpu.make_async_copy(feat_hbm.at[sidx.at[c1]], rows1, sem1).wait()
            pltpu.async_copy(rows1, acc.at[didx.at[c1]], sem3, add=True)
            pltpu.make_async_copy(rows0, acc.at[didx.at[c0]], sem2).wait()

            @pl.when(c0 + 2 < SCHUNKS)
            def _():
                pltpu.async_copy(feat_hbm.at[sidx.at[c0 + 2]], rows0, sem0)

            pltpu.make_async_copy(rows1, acc.at[didx.at[c1]], sem3).wait()

            @pl.when(c1 + 2 < SCHUNKS)
            def _():
                pltpu.async_copy(feat_hbm.at[sidx.at[c1 + 2]], rows1, sem1)
            return ()

        lax.fori_loop(0, SCHUNKS // 2, body, ())
    plsc.subcore_barrier()

    def fbody(t, _):
        r0 = sid * STRIPE + t * FCH
        pltpu.sync_copy(acc.at[pl.ds(r0, FCH)], rows0.at[pl.ds(0, FCH)])
        pltpu.sync_copy(rows0.at[pl.ds(0, FCH)],
                        out_hbm.at[pl.ds(cid * N_PAD + r0, FCH)])
        return ()

    lax.fori_loop(0, FLUSH, fbody, ())


def _combine_body(eps_ref, feat_ref, p0_ref, p1_ref, out_ref):
    out_ref[...] = ((1.0 + eps_ref[0]) * feat_ref[...]
                    + p0_ref[...] + p1_ref[...])


_R = 80


def _combine(eps, feat, partials):
    return pl.pallas_call(
        _combine_body,
        grid=(N_NODES // _R,),
        in_specs=[
            pl.BlockSpec(memory_space=pltpu.SMEM),
            pl.BlockSpec((_R, D), lambda i: (i, 0)),
            pl.BlockSpec((_R, D), lambda i: (i, 0)),
            pl.BlockSpec((_R, D), lambda i: (i + N_PAD // _R, 0)),
        ],
        out_specs=pl.BlockSpec((_R, D), lambda i: (i, 0)),
        out_shape=jax.ShapeDtypeStruct((N_NODES, D), jnp.float32),
    )(eps, feat, partials, partials)


def kernel(edge_index, split_list, feat, eps):
    src = edge_index[0].astype(jnp.int32).reshape(NW, N_STAGES, SCHUNKS, CHUNK)
    dst = edge_index[1].astype(jnp.int32).reshape(NW, N_STAGES, SCHUNKS, CHUNK)
    partials = _gin_scatter(src, dst, feat)
    return _combine(eps, feat, partials)


# speedup vs baseline: 1.2019x; 1.2019x over previous
import functools

import jax
import jax.numpy as jnp
from jax import lax
from jax.experimental import pallas as pl
from jax.experimental.pallas import tpu as pltpu
from jax.experimental.pallas import tpu_sc as plsc

N_NODES = 10000
D = 128
E = 320000
NW = 32
E_PER_W = E // NW
CHUNK = 125
N_CHUNKS = E_PER_W // CHUNK
N_STAGES = 4
SCHUNKS = N_CHUNKS // N_STAGES
N_PAD = 10240
STRIPE = N_PAD // 16
FCH = 80
FLUSH = STRIPE // FCH

_mesh = plsc.VectorSubcoreMesh(core_axis_name="c", subcore_axis_name="s")


@functools.partial(
    pl.kernel,
    out_type=jax.ShapeDtypeStruct((2 * N_PAD, D), jnp.float32),
    mesh=_mesh,
    scratch_types=[
        pltpu.VMEM((SCHUNKS, CHUNK), jnp.int32),
        pltpu.VMEM((SCHUNKS, CHUNK), jnp.int32),
        pltpu.VMEM((CHUNK, D), jnp.float32),
        pltpu.VMEM((CHUNK, D), jnp.float32),
        pltpu.VMEM_SHARED((N_PAD, D), jnp.float32),
        pltpu.SemaphoreType.DMA,
        pltpu.SemaphoreType.DMA,
        pltpu.SemaphoreType.DMA,
        pltpu.SemaphoreType.DMA,
    ],
)
def _gin_scatter(src_hbm, dst_hbm, feat_hbm, out_hbm,
                 sidx, didx, rows0, rows1, acc, sem0, sem1, sem2, sem3):
    cid = lax.axis_index("c")
    sid = lax.axis_index("s")
    wid = cid * 16 + sid

    zero = jnp.zeros((16,), jnp.float32)

    def zrow(r, _):
        def zcol(c, _):
            rows0[r, pl.ds(c * 16, 16)] = zero
            return ()
        lax.fori_loop(0, D // 16, zcol, ())
        return ()

    lax.fori_loop(0, FCH, zrow, ())

    zsrc = rows0.at[pl.ds(0, FCH)]
    for t in range(FLUSH):
        pltpu.async_copy(zsrc, acc.at[pl.ds(sid * STRIPE + t * FCH, FCH)], sem0)
    for t in range(FLUSH):
        pltpu.make_async_copy(
            zsrc, acc.at[pl.ds(sid * STRIPE + t * FCH, FCH)], sem0).wait()
    plsc.subcore_barrier()

    for s in range(N_STAGES):
        pltpu.sync_copy(src_hbm.at[wid, s], sidx)
        pltpu.sync_copy(dst_hbm.at[wid, s], didx)
        pltpu.async_copy(feat_hbm.at[sidx.at[0]], rows0, sem0)
        pltpu.async_copy(feat_hbm.at[sidx.at[1]], rows1, sem1)

        def body(j, _):
            c0 = 2 * j
            c1 = c0 + 1
            pltpu.make_async_copy(feat_hbm.at[sidx.at[c0]], rows0, sem0).wait()
            pltpu.sync_copy(rows0, acc.at[didx.at[c0]], add=True)

            @pl.when(c0 + 2 < SCHUNKS)
            def _():
                pltpu.async_copy(feat_hbm.at[sidx.at[c0 + 2]], rows0, sem0)

            pltpu.make_async_copy(feat_hbm.at[sidx.at[c1]], rows1, sem1).wait()
            pltpu.sync_copy(rows1, acc.at[didx.at[c1]], add=True)

            @pl.when(c1 + 2 < SCHUNKS)
            def _():
                pltpu.async_copy(feat_hbm.at[sidx.at[c1 + 2]], rows1, sem1)
            return ()

        lax.fori_loop(0, SCHUNKS // 2, body, ())
    plsc.subcore_barrier()

    def fsrc(t):
        return acc.at[pl.ds(sid * STRIPE + t * FCH, FCH)]

    def fdst(t):
        return out_hbm.at[pl.ds(cid * N_PAD + sid * STRIPE + t * FCH, FCH)]

    fbufs = (rows0.at[pl.ds(0, FCH)], rows1.at[pl.ds(0, FCH)])
    fsems = (sem0, sem1)
    for t in range(FLUSH):
        buf, sem = fbufs[t % 2], fsems[t % 2]
        if t >= 2:
            pltpu.make_async_copy(buf, fdst(t - 2), sem).wait()
        pltpu.sync_copy(fsrc(t), buf)
        pltpu.async_copy(buf, fdst(t), sem)
    for t in range(FLUSH - 2, FLUSH):
        buf, sem = fbufs[t % 2], fsems[t % 2]
        pltpu.make_async_copy(buf, fdst(t), sem).wait()


def _combine_body(eps_ref, feat_ref, p0_ref, p1_ref, out_ref):
    out_ref[...] = ((1.0 + eps_ref[0]) * feat_ref[...]
                    + p0_ref[...] + p1_ref[...])


_R = 80


def _combine(eps, feat, partials):
    return pl.pallas_call(
        _combine_body,
        grid=(N_NODES // _R,),
        in_specs=[
            pl.BlockSpec(memory_space=pltpu.SMEM),
            pl.BlockSpec((_R, D), lambda i: (i, 0)),
            pl.BlockSpec((_R, D), lambda i: (i, 0)),
            pl.BlockSpec((_R, D), lambda i: (i + N_PAD // _R, 0)),
        ],
        out_specs=pl.BlockSpec((_R, D), lambda i: (i, 0)),
        out_shape=jax.ShapeDtypeStruct((N_NODES, D), jnp.float32),
    )(eps, feat, partials, partials)


def kernel(edge_index, split_list, feat, eps):
    src = edge_index[0].astype(jnp.int32).reshape(NW, N_STAGES, SCHUNKS, CHUNK)
    dst = edge_index[1].astype(jnp.int32).reshape(NW, N_STAGES, SCHUNKS, CHUNK)
    partials = _gin_scatter(src, dst, feat)
    return _combine(eps, feat, partials)


# speedup vs baseline: 1.2020x; 1.0001x over previous
import functools

import jax
import jax.numpy as jnp
from jax import lax
from jax.experimental import pallas as pl
from jax.experimental.pallas import tpu as pltpu
from jax.experimental.pallas import tpu_sc as plsc

N_NODES = 10000
D = 128
E = 320000
NW = 32
E_PER_W = E // NW
CHUNK = 50
N_CHUNKS = E_PER_W // CHUNK
N_STAGES = 5
SCHUNKS = N_CHUNKS // N_STAGES
NBUF = 4
N_PAD = 10240
STRIPE = N_PAD // 16
FCH = 40
FLUSH = STRIPE // FCH

_mesh = plsc.VectorSubcoreMesh(core_axis_name="c", subcore_axis_name="s")


@functools.partial(
    pl.kernel,
    out_type=jax.ShapeDtypeStruct((2 * N_PAD, D), jnp.float32),
    mesh=_mesh,
    scratch_types=[
        pltpu.VMEM((SCHUNKS, CHUNK), jnp.int32),
        pltpu.VMEM((SCHUNKS, CHUNK), jnp.int32),
        [pltpu.VMEM((CHUNK, D), jnp.float32)] * NBUF,
        pltpu.VMEM_SHARED((N_PAD, D), jnp.float32),
        [pltpu.SemaphoreType.DMA] * NBUF,
    ],
)
def _gin_scatter(src_hbm, dst_hbm, feat_hbm, out_hbm,
                 sidx, didx, rows, acc, sems):
    cid = lax.axis_index("c")
    sid = lax.axis_index("s")
    wid = cid * 16 + sid

    zero = jnp.zeros((16,), jnp.float32)

    def zrow(r, _):
        def zcol(c, _):
            rows[0][r, pl.ds(c * 16, 16)] = zero
            return ()
        lax.fori_loop(0, D // 16, zcol, ())
        return ()

    lax.fori_loop(0, FCH, zrow, ())

    zsrc = rows[0].at[pl.ds(0, FCH)]
    for t in range(FLUSH):
        dst = acc.at[pl.ds(sid * STRIPE + t * FCH, FCH)]
        pltpu.async_copy(zsrc, dst, sems[0])
    for t in range(FLUSH):
        dst = acc.at[pl.ds(sid * STRIPE + t * FCH, FCH)]
        pltpu.make_async_copy(zsrc, dst, sems[0]).wait()
    plsc.subcore_barrier()

    for s in range(N_STAGES):
        pltpu.sync_copy(src_hbm.at[wid, s], sidx)
        pltpu.sync_copy(dst_hbm.at[wid, s], didx)
        for b in range(NBUF):
            pltpu.async_copy(feat_hbm.at[sidx.at[b]], rows[b], sems[b])

        def body(j, _):
            c0 = NBUF * j
            for b in range(NBUF):
                c = c0 + b
                pltpu.make_async_copy(
                    feat_hbm.at[sidx.at[c]], rows[b], sems[b]).wait()
                pltpu.sync_copy(rows[b], acc.at[didx.at[c]], add=True)

                @pl.when(c + NBUF < SCHUNKS)
                def _():
                    pltpu.async_copy(
                        feat_hbm.at[sidx.at[c + NBUF]], rows[b], sems[b])
            return ()

        lax.fori_loop(0, SCHUNKS // NBUF, body, ())
    plsc.subcore_barrier()

    def fsrc(t):
        return acc.at[pl.ds(sid * STRIPE + t * FCH, FCH)]

    def fdst(t):
        return out_hbm.at[pl.ds(cid * N_PAD + sid * STRIPE + t * FCH, FCH)]

    fbufs = (rows[0].at[pl.ds(0, FCH)], rows[1].at[pl.ds(0, FCH)])
    fsems = (sems[0], sems[1])
    for t in range(FLUSH):
        buf, sem = fbufs[t % 2], fsems[t % 2]
        if t >= 2:
            pltpu.make_async_copy(buf, fdst(t - 2), sem).wait()
        pltpu.sync_copy(fsrc(t), buf)
        pltpu.async_copy(buf, fdst(t), sem)
    for t in range(FLUSH - 2, FLUSH):
        buf, sem = fbufs[t % 2], fsems[t % 2]
        pltpu.make_async_copy(buf, fdst(t), sem).wait()


def _combine_body(eps_ref, feat_ref, p0_ref, p1_ref, out_ref):
    out_ref[...] = ((1.0 + eps_ref[0]) * feat_ref[...]
                    + p0_ref[...] + p1_ref[...])


_R = 80


def _combine(eps, feat, partials):
    return pl.pallas_call(
        _combine_body,
        grid=(N_NODES // _R,),
        in_specs=[
            pl.BlockSpec(memory_space=pltpu.SMEM),
            pl.BlockSpec((_R, D), lambda i: (i, 0)),
            pl.BlockSpec((_R, D), lambda i: (i, 0)),
            pl.BlockSpec((_R, D), lambda i: (i + N_PAD // _R, 0)),
        ],
        out_specs=pl.BlockSpec((_R, D), lambda i: (i, 0)),
        out_shape=jax.ShapeDtypeStruct((N_NODES, D), jnp.float32),
    )(eps, feat, partials, partials)


def kernel(edge_index, split_list, feat, eps):
    src = edge_index[0].astype(jnp.int32).reshape(NW, N_STAGES, SCHUNKS, CHUNK)
    dst = edge_index[1].astype(jnp.int32).reshape(NW, N_STAGES, SCHUNKS, CHUNK)
    partials = _gin_scatter(src, dst, feat)
    return _combine(eps, feat, partials)


# speedup vs baseline: 1.2112x; 1.0077x over previous
import functools

import jax
import jax.numpy as jnp
from jax import lax
from jax.experimental import pallas as pl
from jax.experimental.pallas import tpu as pltpu
from jax.experimental.pallas import tpu_sc as plsc

N_NODES = 10000
D = 128
E = 320000
NW = 32
E_PER_W = E // NW
CHUNK = 50
N_CHUNKS = E_PER_W // CHUNK
N_STAGES = 5
SCHUNKS = N_CHUNKS // N_STAGES
NBUF = 4
N_PAD = 10240
STRIPE = N_PAD // 16
FCH = 40
FLUSH = STRIPE // FCH

_mesh = plsc.VectorSubcoreMesh(core_axis_name="c", subcore_axis_name="s")


@functools.partial(
    pl.kernel,
    out_type=jax.ShapeDtypeStruct((2 * N_PAD, D), jnp.float32),
    mesh=_mesh,
    scratch_types=[
        pltpu.VMEM((SCHUNKS, CHUNK), jnp.int32),
        pltpu.VMEM((SCHUNKS, CHUNK), jnp.int32),
        [pltpu.VMEM((CHUNK, D), jnp.float32)] * NBUF,
        pltpu.VMEM((FCH, D), jnp.float32),
        pltpu.VMEM_SHARED((N_PAD, D), jnp.float32),
        [pltpu.SemaphoreType.DMA] * (NBUF + 1),
    ],
)
def _gin_scatter(src_hbm, dst_hbm, feat_hbm, out_hbm,
                 sidx, didx, rows, zbuf, acc, sems):
    cid = lax.axis_index("c")
    sid = lax.axis_index("s")
    wid = cid * 16 + sid
    zsem = sems[NBUF]

    pltpu.async_copy(src_hbm.at[wid, 0], sidx, sems[0])
    pltpu.async_copy(dst_hbm.at[wid, 0], didx, sems[1])

    zero = jnp.zeros((16,), jnp.float32)

    def zrow(r, _):
        def zcol(c, _):
            zbuf[r, pl.ds(c * 16, 16)] = zero
            return ()
        lax.fori_loop(0, D // 16, zcol, ())
        return ()

    lax.fori_loop(0, FCH, zrow, ())

    for t in range(FLUSH):
        pltpu.async_copy(zbuf, acc.at[pl.ds(sid * STRIPE + t * FCH, FCH)], zsem)

    pltpu.make_async_copy(src_hbm.at[wid, 0], sidx, sems[0]).wait()
    pltpu.make_async_copy(dst_hbm.at[wid, 0], didx, sems[1]).wait()
    for b in range(NBUF):
        pltpu.async_copy(feat_hbm.at[sidx.at[b]], rows[b], sems[b])
    for t in range(FLUSH):
        pltpu.make_async_copy(
            zbuf, acc.at[pl.ds(sid * STRIPE + t * FCH, FCH)], zsem).wait()
    plsc.subcore_barrier()

    for s in range(N_STAGES):
        if s > 0:
            pltpu.sync_copy(src_hbm.at[wid, s], sidx)
            pltpu.sync_copy(dst_hbm.at[wid, s], didx)
            for b in range(NBUF):
                pltpu.async_copy(feat_hbm.at[sidx.at[b]], rows[b], sems[b])

        def body(j, _):
            c0 = NBUF * j
            for b in range(NBUF):
                c = c0 + b
                pltpu.make_async_copy(
                    feat_hbm.at[sidx.at[c]], rows[b], sems[b]).wait()
                pltpu.sync_copy(rows[b], acc.at[didx.at[c]], add=True)

                @pl.when(c + NBUF < SCHUNKS)
                def _():
                    pltpu.async_copy(
                        feat_hbm.at[sidx.at[c + NBUF]], rows[b], sems[b])
            return ()

        lax.fori_loop(0, SCHUNKS // NBUF, body, ())
    plsc.subcore_barrier()

    def fsrc(t):
        return acc.at[pl.ds(sid * STRIPE + t * FCH, FCH)]

    def fdst(t):
        return out_hbm.at[pl.ds(cid * N_PAD + sid * STRIPE + t * FCH, FCH)]

    for t in range(FLUSH):
        pltpu.async_copy(fsrc(t), fdst(t), zsem)
    for t in range(FLUSH):
        pltpu.make_async_copy(fsrc(t), fdst(t), zsem).wait()


def _combine_body(eps_ref, feat_ref, p0_ref, p1_ref, out_ref):
    out_ref[...] = ((1.0 + eps_ref[0]) * feat_ref[...]
                    + p0_ref[...] + p1_ref[...])


_R = 80


def _combine(eps, feat, partials):
    return pl.pallas_call(
        _combine_body,
        grid=(N_NODES // _R,),
        in_specs=[
            pl.BlockSpec(memory_space=pltpu.SMEM),
            pl.BlockSpec((_R, D), lambda i: (i, 0)),
            pl.BlockSpec((_R, D), lambda i: (i, 0)),
            pl.BlockSpec((_R, D), lambda i: (i + N_PAD // _R, 0)),
        ],
        out_specs=pl.BlockSpec((_R, D), lambda i: (i, 0)),
        out_shape=jax.ShapeDtypeStruct((N_NODES, D), jnp.float32),
    )(eps, feat, partials, partials)


def kernel(edge_index, split_list, feat, eps):
    src = edge_index[0].astype(jnp.int32).reshape(NW, N_STAGES, SCHUNKS, CHUNK)
    dst = edge_index[1].astype(jnp.int32).reshape(NW, N_STAGES, SCHUNKS, CHUNK)
    partials = _gin_scatter(src, dst, feat)
    return _combine(eps, feat, partials)
